# Initial kernel scaffold; baseline (speedup 1.0000x reference)
import jax
import jax.numpy as jnp
from jax.experimental import pallas as pl


def kernel(features, edge_index, W1, b1, W2, b2, W3, b3):
    raise NotImplementedError("write your pallas kernel here")


# speedup vs baseline: 13.4913x; 13.4913x over previous
import functools

import jax
import jax.numpy as jnp
from jax import lax
from jax.experimental import pallas as pl
from jax.experimental.pallas import tpu as pltpu
from jax.experimental.pallas import tpu_sc as plsc

N = 100000
E = 1600000
F = 30
NC = 2
NS = 16
SUB = 128
STRIPE = 6256
N_PAD = NS * STRIPE

R_T2 = 784
E2 = NS * R_T2 * SUB
K2 = 8
NCH2 = R_T2 // K2
CH2 = K2 * SUB

R_T1 = R_T2 // NC
K1 = 8
NCH1 = R_T1 // K1
CH1 = K1 * SUB

_mesh = plsc.VectorSubcoreMesh(core_axis_name="c", subcore_axis_name="s")


@functools.partial(
    pl.kernel,
    out_type=jax.ShapeDtypeStruct((NC * N_PAD,), jnp.float32),
    mesh=_mesh,
    compiler_params=pltpu.CompilerParams(use_tc_tiling_on_sc=False),
    scratch_types=[
        pltpu.VMEM((K1, SUB), jnp.int32),
        pltpu.VMEM((CH1,), jnp.float32),
        pltpu.VMEM((STRIPE,), jnp.float32),
        pltpu.VMEM_SHARED((N_PAD,), jnp.float32),
        pltpu.SemaphoreType.DMA,
    ],
)
def _deg_kernel(src2d, out, idx_v, ones_v, zbuf, deg_sh, sem):
    c = lax.axis_index("c")
    s = lax.axis_index("s")

    def zbody(i, _):
        zbuf[pl.ds(i * 16, 16)] = jnp.zeros((16,), jnp.float32)
        return 0
    lax.fori_loop(0, STRIPE // 16, zbody, 0)

    def obody(i, _):
        ones_v[pl.ds(i * 16, 16)] = jnp.ones((16,), jnp.float32)
        return 0
    lax.fori_loop(0, CH1 // 16, obody, 0)

    pltpu.sync_copy(zbuf, deg_sh.at[pl.ds(s * STRIPE, STRIPE)])
    plsc.subcore_barrier()

    row0 = c * (NS * R_T1) + s * R_T1
    def chunk_body(k, _):
        pltpu.sync_copy(src2d.at[pl.ds(row0 + k * K1, K1)], idx_v)
        for j in range(K1):
            pltpu.sync_copy(ones_v.at[pl.ds(j * SUB, SUB)],
                            deg_sh.at[idx_v.at[j]], add=True)
        return 0
    lax.fori_loop(0, NCH1, chunk_body, 0)
    plsc.subcore_barrier()
    pltpu.sync_copy(deg_sh.at[pl.ds(s * STRIPE, STRIPE)], zbuf)
    pltpu.sync_copy(zbuf, out.at[pl.ds(c * N_PAD + s * STRIPE, STRIPE)])


@functools.partial(
    pl.kernel,
    out_type=jax.ShapeDtypeStruct((NC, N_PAD, 16), jnp.float32),
    mesh=_mesh,
    compiler_params=pltpu.CompilerParams(use_tc_tiling_on_sc=False),
    scratch_types=[
        pltpu.VMEM((CH2,), jnp.int32),
        pltpu.VMEM((K2, SUB), jnp.int32),
        pltpu.VMEM((CH2, 16), jnp.float32),
        pltpu.VMEM_SHARED((N_PAD, 16), jnp.float32),
        pltpu.SemaphoreType.DMA,
    ],
)
def _agg_kernel(tables, src1d, dst2d, out, sidx_v, didx_v, rows_v, agg_sh, sem):
    c = lax.axis_index("c")
    s = lax.axis_index("s")

    def zrow(i, _):
        rows_v[i] = jnp.zeros((16,), jnp.float32)
        return 0
    lax.fori_loop(0, CH2, zrow, 0)
    nfull = STRIPE // CH2
    tail = STRIPE - nfull * CH2
    for t in range(nfull):
        pltpu.sync_copy(rows_v, agg_sh.at[pl.ds(s * STRIPE + t * CH2, CH2)])
    pltpu.sync_copy(rows_v.at[pl.ds(0, tail)],
                    agg_sh.at[pl.ds(s * STRIPE + nfull * CH2, tail)])
    plsc.subcore_barrier()

    base = s * R_T2 * SUB
    row_base = s * R_T2
    def chunk_body(k, _):
        pltpu.sync_copy(src1d.at[pl.ds(base + k * CH2, CH2)], sidx_v)
        pltpu.sync_copy(dst2d.at[pl.ds(row_base + k * K2, K2)], didx_v)
        pltpu.async_copy(tables.at[c].at[sidx_v], rows_v, sem).wait()
        for j in range(K2):
            pltpu.sync_copy(rows_v.at[pl.ds(j * SUB, SUB)],
                            agg_sh.at[didx_v.at[j]], add=True)
        return 0
    lax.fori_loop(0, NCH2, chunk_body, 0)
    plsc.subcore_barrier()
    for t in range(nfull):
        pltpu.sync_copy(agg_sh.at[pl.ds(s * STRIPE + t * CH2, CH2)], rows_v)
        pltpu.sync_copy(rows_v, out.at[c, pl.ds(s * STRIPE + t * CH2, CH2)])
    pltpu.sync_copy(agg_sh.at[pl.ds(s * STRIPE + nfull * CH2, tail)],
                    rows_v.at[pl.ds(0, tail)])
    pltpu.sync_copy(rows_v.at[pl.ds(0, tail)],
                    out.at[c, pl.ds(s * STRIPE + nfull * CH2, tail)])


_XB = 2000
_XG = N // _XB


def _xbuild_body(f_ref, d0_ref, d1_ref, out_ref):
    scale = lax.rsqrt(jnp.maximum(d0_ref[...] + d1_ref[...], 1.0))
    x = f_ref[...] * scale
    ones = jnp.ones((_XB, 1), jnp.float32)
    zeros = jnp.zeros((_XB, 1), jnp.float32)
    out_ref[0] = jnp.concatenate([x[:, :15], ones], axis=1)
    out_ref[1] = jnp.concatenate([x[:, 15:], zeros], axis=1)


def _silu(x):
    return x * jax.nn.sigmoid(x)


def _head_body(agg_ref, w1_ref, b1_ref, w2_ref, b2_ref, w3_ref, b3_ref,
               out_ref, cmin_ref, cmax_ref):
    i = pl.program_id(0)

    @pl.when(i == 0)
    def _():
        cmin_ref[...] = jnp.full((1, 500), jnp.inf, jnp.float32)
        cmax_ref[...] = jnp.full((1, 500), -jnp.inf, jnp.float32)

    a = agg_ref[0]
    b = agg_ref[1]
    scale = lax.rsqrt(jnp.maximum(a[:, 15:16], 1.0))
    ab = jnp.concatenate([a * scale, b * scale], axis=1)
    h = jnp.dot(ab, w1_ref[...], preferred_element_type=jnp.float32)
    h = h + b1_ref[...]
    cmin_ref[...] = jnp.minimum(cmin_ref[...], jnp.min(h, axis=0, keepdims=True))
    cmax_ref[...] = jnp.maximum(cmax_ref[...], jnp.max(h, axis=0, keepdims=True))

    @pl.when(i == _XG - 1)
    def _():
        pooled = jnp.maximum(_silu(cmin_ref[...]), _silu(cmax_ref[...]))
        z = _silu(jnp.dot(pooled, w2_ref[...],
                          preferred_element_type=jnp.float32) + b2_ref[...])
        out_ref[...] = jax.nn.sigmoid(
            jnp.dot(z, w3_ref[...], preferred_element_type=jnp.float32)
            + b3_ref[...])


def kernel(features, edge_index, W1, b1, W2, b2, W3, b3):
    pad = jnp.full((E2 - E,), N, jnp.int32)
    src = jnp.concatenate([edge_index[0], pad])
    dst = jnp.concatenate([edge_index[1], pad])
    src2d = src.reshape(E2 // SUB, SUB)
    dst2d = dst.reshape(E2 // SUB, SUB)

    degs = _deg_kernel(src2d)
    d0 = degs[:N].reshape(N, 1)
    d1 = degs[N_PAD:N_PAD + N].reshape(N, 1)

    xbuild = pl.pallas_call(
        _xbuild_body,
        grid=(_XG,),
        in_specs=[
            pl.BlockSpec((_XB, F), lambda i: (i, 0)),
            pl.BlockSpec((_XB, 1), lambda i: (i, 0)),
            pl.BlockSpec((_XB, 1), lambda i: (i, 0)),
        ],
        out_specs=pl.BlockSpec((NC, _XB, 16), lambda i: (0, i, 0)),
        out_shape=jax.ShapeDtypeStruct((NC, N_PAD, 16), jnp.float32),
    )
    tables = xbuild(features, d0, d1)

    agg = _agg_kernel(tables, src, dst2d)

    W1p = jnp.zeros((32, 500), jnp.float32)
    W1p = W1p.at[0:15].set(W1[0:15])
    W1p = W1p.at[16:31].set(W1[15:30])

    head = pl.pallas_call(
        _head_body,
        grid=(_XG,),
        in_specs=[
            pl.BlockSpec((NC, _XB, 16), lambda i: (0, i, 0)),
            pl.BlockSpec((32, 500), lambda i: (0, 0)),
            pl.BlockSpec((1, 500), lambda i: (0, 0)),
            pl.BlockSpec((500, 20), lambda i: (0, 0)),
            pl.BlockSpec((1, 20), lambda i: (0, 0)),
            pl.BlockSpec((20, 4), lambda i: (0, 0)),
            pl.BlockSpec((1, 4), lambda i: (0, 0)),
        ],
        out_specs=pl.BlockSpec((1, 4), lambda i: (0, 0)),
        out_shape=jax.ShapeDtypeStruct((1, 4), jnp.float32),
        scratch_shapes=[
            pltpu.VMEM((1, 500), jnp.float32),
            pltpu.VMEM((1, 500), jnp.float32),
        ],
    )
    return head(agg, W1p, b1.reshape(1, 500), W2, b2.reshape(1, 20),
                W3, b3.reshape(1, 4))


# speedup vs baseline: 15.4379x; 1.1443x over previous
import functools

import jax
import jax.numpy as jnp
from jax import lax
from jax.experimental import pallas as pl
from jax.experimental.pallas import tpu as pltpu
from jax.experimental.pallas import tpu_sc as plsc

N = 100000
E = 1600000
F = 30
NC = 2
NS = 16
SUB = 128
STRIPE = 6256
N_PAD = NS * STRIPE

R_T2 = 784
E2 = NS * R_T2 * SUB
K2 = 4
NCH2 = R_T2 // K2
CH2 = K2 * SUB
NLOOP2 = NCH2 // 2

R_T1 = R_T2 // NC
K1 = 7
NCH1 = R_T1 // K1
CH1 = K1 * SUB
NLOOP1 = NCH1 // 2

_mesh = plsc.VectorSubcoreMesh(core_axis_name="c", subcore_axis_name="s")


@functools.partial(
    pl.kernel,
    out_type=jax.ShapeDtypeStruct((NC * N_PAD,), jnp.float32),
    mesh=_mesh,
    compiler_params=pltpu.CompilerParams(use_tc_tiling_on_sc=False),
    scratch_types=[
        pltpu.VMEM((K1, SUB), jnp.int32),
        pltpu.VMEM((K1, SUB), jnp.int32),
        pltpu.VMEM((SUB,), jnp.float32),
        pltpu.VMEM((STRIPE,), jnp.float32),
        pltpu.VMEM_SHARED((N_PAD,), jnp.float32),
        pltpu.SemaphoreType.DMA,
        pltpu.SemaphoreType.DMA,
    ],
)
def _deg_kernel(src2d, out, idxA, idxB, ones_v, zbuf, deg_sh, lsem, ssem):
    c = lax.axis_index("c")
    s = lax.axis_index("s")

    def zbody(i, _):
        zbuf[pl.ds(i * 16, 16)] = jnp.zeros((16,), jnp.float32)
        return 0
    lax.fori_loop(0, STRIPE // 16, zbody, 0)
    for i in range(SUB // 16):
        ones_v[pl.ds(i * 16, 16)] = jnp.ones((16,), jnp.float32)
    pltpu.sync_copy(zbuf, deg_sh.at[pl.ds(s * STRIPE, STRIPE)])
    plsc.subcore_barrier()

    row0 = c * (NS * R_T1) + s * R_T1

    def scatter_chunk(idx):
        descs = [pltpu.async_copy(ones_v, deg_sh.at[idx.at[j]], ssem, add=True)
                 for j in range(K1)]
        for d in descs:
            d.wait()

    pltpu.sync_copy(src2d.at[pl.ds(row0, K1)], idxA)

    def body(k, _):
        lb = pltpu.async_copy(src2d.at[pl.ds(row0 + (2 * k + 1) * K1, K1)],
                              idxB, lsem)
        scatter_chunk(idxA)
        lb.wait()

        @pl.when(k < NLOOP1 - 1)
        def _():
            pltpu.async_copy(src2d.at[pl.ds(row0 + (2 * k + 2) * K1, K1)],
                             idxA, lsem).wait()
        scatter_chunk(idxB)
        return 0
    lax.fori_loop(0, NLOOP1, body, 0)
    plsc.subcore_barrier()
    pltpu.sync_copy(deg_sh.at[pl.ds(s * STRIPE, STRIPE)], zbuf)
    pltpu.sync_copy(zbuf, out.at[pl.ds(c * N_PAD + s * STRIPE, STRIPE)])


@functools.partial(
    pl.kernel,
    out_type=jax.ShapeDtypeStruct((NC, N_PAD, 16), jnp.float32),
    mesh=_mesh,
    compiler_params=pltpu.CompilerParams(use_tc_tiling_on_sc=False),
    scratch_types=[
        pltpu.VMEM((CH2,), jnp.int32),
        pltpu.VMEM((CH2,), jnp.int32),
        pltpu.VMEM((K2, SUB), jnp.int32),
        pltpu.VMEM((K2, SUB), jnp.int32),
        pltpu.VMEM((CH2, 16), jnp.float32),
        pltpu.VMEM((CH2, 16), jnp.float32),
        pltpu.VMEM_SHARED((N_PAD, 16), jnp.float32),
        pltpu.SemaphoreType.DMA,
        pltpu.SemaphoreType.DMA,
        pltpu.SemaphoreType.DMA,
        pltpu.SemaphoreType.DMA,
    ],
)
def _agg_kernel(tables, src1d, dst2d, out, sA, sB, dA, dB, rA, rB,
                agg_sh, gsA, gsB, lsem, ssem):
    c = lax.axis_index("c")
    s = lax.axis_index("s")

    def zrow(i, _):
        rA[i] = jnp.zeros((16,), jnp.float32)
        return 0
    lax.fori_loop(0, CH2, zrow, 0)
    nfull = STRIPE // CH2
    tail = STRIPE - nfull * CH2
    for t in range(nfull):
        pltpu.sync_copy(rA, agg_sh.at[pl.ds(s * STRIPE + t * CH2, CH2)])
    pltpu.sync_copy(rA.at[pl.ds(0, tail)],
                    agg_sh.at[pl.ds(s * STRIPE + nfull * CH2, tail)])
    plsc.subcore_barrier()

    base = s * R_T2 * SUB
    row_base = s * R_T2
    tab = tables.at[c]

    def load_idx(k, sidx, didx):
        l1 = pltpu.async_copy(src1d.at[pl.ds(base + k * CH2, CH2)], sidx, lsem)
        l2 = pltpu.async_copy(dst2d.at[pl.ds(row_base + k * K2, K2)], didx, lsem)
        return l1, l2

    def wait_gather(rows, gsem):
        pltpu.make_async_copy(tab.at[pl.ds(0, CH2)], rows, gsem).wait()

    def scatter_chunk(rows, didx):
        descs = [pltpu.async_copy(rows.at[pl.ds(j * SUB, SUB)],
                                  agg_sh.at[didx.at[j]], ssem, add=True)
                 for j in range(K2)]
        for d in descs:
            d.wait()

    l1, l2 = load_idx(0, sA, dA)
    l1.wait()
    l2.wait()
    pltpu.async_copy(tab.at[sA], rA, gsA)

    def body(k, _):
        l1, l2 = load_idx(2 * k + 1, sB, dB)
        wait_gather(rA, gsA)
        l1.wait()
        l2.wait()
        pltpu.async_copy(tab.at[sB], rB, gsB)
        scatter_chunk(rA, dA)

        wait_gather(rB, gsB)

        @pl.when(k < NLOOP2 - 1)
        def _():
            l1, l2 = load_idx(2 * k + 2, sA, dA)
            l1.wait()
            l2.wait()
            pltpu.async_copy(tab.at[sA], rA, gsA)
        scatter_chunk(rB, dB)
        return 0
    lax.fori_loop(0, NLOOP2, body, 0)
    plsc.subcore_barrier()
    for t in range(nfull):
        pltpu.sync_copy(agg_sh.at[pl.ds(s * STRIPE + t * CH2, CH2)], rA)
        pltpu.sync_copy(rA, out.at[c, pl.ds(s * STRIPE + t * CH2, CH2)])
    pltpu.sync_copy(agg_sh.at[pl.ds(s * STRIPE + nfull * CH2, tail)],
                    rA.at[pl.ds(0, tail)])
    pltpu.sync_copy(rA.at[pl.ds(0, tail)],
                    out.at[c, pl.ds(s * STRIPE + nfull * CH2, tail)])


_XB = 2000
_XG = N // _XB


def _xbuild_body(f_ref, d0_ref, d1_ref, out_ref):
    scale = lax.rsqrt(jnp.maximum(d0_ref[...] + d1_ref[...], 1.0))
    x = f_ref[...] * scale
    ones = jnp.ones((_XB, 1), jnp.float32)
    zeros = jnp.zeros((_XB, 1), jnp.float32)
    out_ref[0] = jnp.concatenate([x[:, :15], ones], axis=1)
    out_ref[1] = jnp.concatenate([x[:, 15:], zeros], axis=1)


def _silu(x):
    return x * jax.nn.sigmoid(x)


def _head_body(agg_ref, w1_ref, b1_ref, w2_ref, b2_ref, w3_ref, b3_ref,
               out_ref, cmin_ref, cmax_ref):
    i = pl.program_id(0)

    @pl.when(i == 0)
    def _():
        cmin_ref[...] = jnp.full((1, 500), jnp.inf, jnp.float32)
        cmax_ref[...] = jnp.full((1, 500), -jnp.inf, jnp.float32)

    a = agg_ref[0]
    b = agg_ref[1]
    scale = lax.rsqrt(jnp.maximum(a[:, 15:16], 1.0))
    ab = jnp.concatenate([a * scale, b * scale], axis=1)
    h = jnp.dot(ab, w1_ref[...], preferred_element_type=jnp.float32)
    h = h + b1_ref[...]
    cmin_ref[...] = jnp.minimum(cmin_ref[...], jnp.min(h, axis=0, keepdims=True))
    cmax_ref[...] = jnp.maximum(cmax_ref[...], jnp.max(h, axis=0, keepdims=True))

    @pl.when(i == _XG - 1)
    def _():
        pooled = jnp.maximum(_silu(cmin_ref[...]), _silu(cmax_ref[...]))
        z = _silu(jnp.dot(pooled, w2_ref[...],
                          preferred_element_type=jnp.float32) + b2_ref[...])
        out_ref[...] = jax.nn.sigmoid(
            jnp.dot(z, w3_ref[...], preferred_element_type=jnp.float32)
            + b3_ref[...])


def kernel(features, edge_index, W1, b1, W2, b2, W3, b3):
    pad = jnp.full((E2 - E,), N, jnp.int32)
    src = jnp.concatenate([edge_index[0], pad])
    dst = jnp.concatenate([edge_index[1], pad])
    src2d = src.reshape(E2 // SUB, SUB)
    dst2d = dst.reshape(E2 // SUB, SUB)

    degs = _deg_kernel(src2d)
    d0 = degs[:N].reshape(N, 1)
    d1 = degs[N_PAD:N_PAD + N].reshape(N, 1)

    xbuild = pl.pallas_call(
        _xbuild_body,
        grid=(_XG,),
        in_specs=[
            pl.BlockSpec((_XB, F), lambda i: (i, 0)),
            pl.BlockSpec((_XB, 1), lambda i: (i, 0)),
            pl.BlockSpec((_XB, 1), lambda i: (i, 0)),
        ],
        out_specs=pl.BlockSpec((NC, _XB, 16), lambda i: (0, i, 0)),
        out_shape=jax.ShapeDtypeStruct((NC, N_PAD, 16), jnp.float32),
    )
    tables = xbuild(features, d0, d1)

    agg = _agg_kernel(tables, src, dst2d)

    W1p = jnp.zeros((32, 500), jnp.float32)
    W1p = W1p.at[0:15].set(W1[0:15])
    W1p = W1p.at[16:31].set(W1[15:30])

    head = pl.pallas_call(
        _head_body,
        grid=(_XG,),
        in_specs=[
            pl.BlockSpec((NC, _XB, 16), lambda i: (0, i, 0)),
            pl.BlockSpec((32, 500), lambda i: (0, 0)),
            pl.BlockSpec((1, 500), lambda i: (0, 0)),
            pl.BlockSpec((500, 20), lambda i: (0, 0)),
            pl.BlockSpec((1, 20), lambda i: (0, 0)),
            pl.BlockSpec((20, 4), lambda i: (0, 0)),
            pl.BlockSpec((1, 4), lambda i: (0, 0)),
        ],
        out_specs=pl.BlockSpec((1, 4), lambda i: (0, 0)),
        out_shape=jax.ShapeDtypeStruct((1, 4), jnp.float32),
        scratch_shapes=[
            pltpu.VMEM((1, 500), jnp.float32),
            pltpu.VMEM((1, 500), jnp.float32),
        ],
    )
    return head(agg, W1p, b1.reshape(1, 500), W2, b2.reshape(1, 20),
                W3, b3.reshape(1, 4))


# speedup vs baseline: 17.4521x; 1.1305x over previous
import functools

import jax
import jax.numpy as jnp
from jax import lax
from jax.experimental import pallas as pl
from jax.experimental.pallas import tpu as pltpu
from jax.experimental.pallas import tpu_sc as plsc

N = 100000
E = 1600000
F = 30
NC = 2
NS = 16
SUB = 128
STRIPE = 6400
N_PAD = NS * STRIPE
PROWS = N_PAD * 16 // 128

E_T2 = 100352
E2 = NS * E_T2
ER = E2 // SUB
R_T2 = E_T2 // SUB
K2 = 4
CH2 = K2 * SUB
NCH2 = R_T2 // K2
NLOOP2 = NCH2 // 2

E_T1 = E2 // (NC * NS)
R_T1 = E_T1 // SUB
K1 = 7
CH1 = K1 * SUB
NCH1 = R_T1 // K1
NLOOP1 = NCH1 // 2

_mesh = plsc.VectorSubcoreMesh(core_axis_name="c", subcore_axis_name="s")


@functools.partial(
    pl.kernel,
    out_type=jax.ShapeDtypeStruct((NC * N_PAD,), jnp.float32),
    mesh=_mesh,
    compiler_params=pltpu.CompilerParams(use_tc_tiling_on_sc=False),
    scratch_types=[
        pltpu.VMEM((K1, SUB), jnp.int32),
        pltpu.VMEM((K1, SUB), jnp.int32),
        pltpu.VMEM((SUB,), jnp.float32),
        pltpu.VMEM((STRIPE,), jnp.float32),
        pltpu.VMEM_SHARED((N_PAD,), jnp.float32),
        pltpu.SemaphoreType.DMA,
        pltpu.SemaphoreType.DMA,
    ],
)
def _deg_kernel(edges3, out, idxA, idxB, ones_v, zbuf, deg_sh, lsem, ssem):
    c = lax.axis_index("c")
    s = lax.axis_index("s")

    def zbody(i, _):
        zbuf[pl.ds(i * 16, 16)] = jnp.zeros((16,), jnp.float32)
        return 0
    lax.fori_loop(0, STRIPE // 16, zbody, 0)
    for i in range(SUB // 16):
        ones_v[pl.ds(i * 16, 16)] = jnp.ones((16,), jnp.float32)
    pltpu.sync_copy(zbuf, deg_sh.at[pl.ds(s * STRIPE, STRIPE)])
    plsc.subcore_barrier()

    row0 = (c * NS + s) * R_T1

    def scatter_chunk(idx):
        descs = [pltpu.async_copy(ones_v, deg_sh.at[idx.at[j]], ssem, add=True)
                 for j in range(K1)]
        for d in descs:
            d.wait()

    pltpu.sync_copy(edges3.at[0, pl.ds(row0, K1)], idxA)

    def body(k, _):
        lb = pltpu.async_copy(edges3.at[0, pl.ds(row0 + (2 * k + 1) * K1, K1)],
                              idxB, lsem)
        scatter_chunk(idxA)
        lb.wait()

        @pl.when(k < NLOOP1 - 1)
        def _():
            pltpu.async_copy(edges3.at[0, pl.ds(row0 + (2 * k + 2) * K1, K1)],
                             idxA, lsem).wait()
        scatter_chunk(idxB)
        return 0
    lax.fori_loop(0, NLOOP1, body, 0)
    plsc.subcore_barrier()
    pltpu.sync_copy(deg_sh.at[pl.ds(s * STRIPE, STRIPE)], zbuf)
    pltpu.sync_copy(zbuf, out.at[pl.ds(c * N_PAD + s * STRIPE, STRIPE)])


@functools.partial(
    pl.kernel,
    out_type=jax.ShapeDtypeStruct((NC, N_PAD, 16), jnp.float32),
    mesh=_mesh,
    compiler_params=pltpu.CompilerParams(use_tc_tiling_on_sc=False),
    scratch_types=[
        pltpu.VMEM((CH2,), jnp.int32),
        pltpu.VMEM((CH2,), jnp.int32),
        pltpu.VMEM((K2, SUB), jnp.int32),
        pltpu.VMEM((K2, SUB), jnp.int32),
        pltpu.VMEM((CH2, 16), jnp.float32),
        pltpu.VMEM((CH2, 16), jnp.float32),
        pltpu.VMEM_SHARED((N_PAD, 16), jnp.float32),
        pltpu.SemaphoreType.DMA,
        pltpu.SemaphoreType.DMA,
        pltpu.SemaphoreType.DMA,
        pltpu.SemaphoreType.DMA,
    ],
)
def _agg_kernel(tables, epflat, edges3, out, sA, sB, dA, dB, rA, rB,
                agg_sh, gsA, gsB, lsem, ssem):
    c = lax.axis_index("c")
    s = lax.axis_index("s")

    def zrow(i, _):
        rA[i] = jnp.zeros((16,), jnp.float32)
        return 0
    lax.fori_loop(0, CH2, zrow, 0)
    nfull = STRIPE // CH2
    tail = STRIPE - nfull * CH2
    for t in range(nfull):
        pltpu.sync_copy(rA, agg_sh.at[pl.ds(s * STRIPE + t * CH2, CH2)])
    pltpu.sync_copy(rA.at[pl.ds(0, tail)],
                    agg_sh.at[pl.ds(s * STRIPE + nfull * CH2, tail)])
    plsc.subcore_barrier()

    base = s * E_T2
    row_base = s * R_T2
    tab = tables.at[c]

    def load_idx(k, sidx, didx):
        l1 = pltpu.async_copy(epflat.at[pl.ds(base + k * CH2, CH2)],
                              sidx, lsem)
        l2 = pltpu.async_copy(edges3.at[1, pl.ds(row_base + k * K2, K2)],
                              didx, lsem)
        return l1, l2

    def wait_gather(rows, gsem):
        pltpu.make_async_copy(tab.at[pl.ds(0, CH2)], rows, gsem).wait()

    def scatter_chunk(rows, didx):
        descs = [pltpu.async_copy(rows.at[pl.ds(j * SUB, SUB)],
                                  agg_sh.at[didx.at[j]], ssem, add=True)
                 for j in range(K2)]
        for d in descs:
            d.wait()

    l1, l2 = load_idx(0, sA, dA)
    l1.wait()
    l2.wait()
    pltpu.async_copy(tab.at[sA], rA, gsA)

    def body(k, _):
        l1, l2 = load_idx(2 * k + 1, sB, dB)
        wait_gather(rA, gsA)
        l1.wait()
        l2.wait()
        pltpu.async_copy(tab.at[sB], rB, gsB)
        scatter_chunk(rA, dA)

        wait_gather(rB, gsB)

        @pl.when(k < NLOOP2 - 1)
        def _():
            l1, l2 = load_idx(2 * k + 2, sA, dA)
            l1.wait()
            l2.wait()
            pltpu.async_copy(tab.at[sA], rA, gsA)
        scatter_chunk(rB, dB)
        return 0
    lax.fori_loop(0, NLOOP2, body, 0)
    plsc.subcore_barrier()
    for t in range(nfull):
        pltpu.sync_copy(agg_sh.at[pl.ds(s * STRIPE + t * CH2, CH2)], rA)
        pltpu.sync_copy(rA, out.at[c, pl.ds(s * STRIPE + t * CH2, CH2)])
    pltpu.sync_copy(agg_sh.at[pl.ds(s * STRIPE + nfull * CH2, tail)],
                    rA.at[pl.ds(0, tail)])
    pltpu.sync_copy(rA.at[pl.ds(0, tail)],
                    out.at[c, pl.ds(s * STRIPE + nfull * CH2, tail)])


_XB = 2048
_XG = N_PAD // _XB
_XB1 = 2000
_XG1 = N // _XB1


def _xbuild_body(f_ref, deg_ref, out_ref):
    dT = deg_ref[...]
    d = dT[:, 0:1] + dT[:, 1:2]
    scale = lax.rsqrt(jnp.maximum(d, 1.0))
    x = f_ref[...] * scale
    ones = jnp.ones((_XB1, 1), jnp.float32)
    zeros = jnp.zeros((_XB1, 1), jnp.float32)
    out_ref[0] = jnp.concatenate([x[:, :15], ones], axis=1)
    out_ref[1] = jnp.concatenate([x[:, 15:], zeros], axis=1)


def _silu(x):
    return x * jax.nn.sigmoid(x)


def _head_body(agg_ref, w1_ref, b1_ref, w2_ref, b2_ref, w3_ref, b3_ref,
               out_ref, cmin_ref, cmax_ref):
    i = pl.program_id(0)

    @pl.when(i == 0)
    def _():
        cmin_ref[...] = jnp.full((1, 500), jnp.inf, jnp.float32)
        cmax_ref[...] = jnp.full((1, 500), -jnp.inf, jnp.float32)

    a = agg_ref[0]
    b = agg_ref[1]
    scale = lax.rsqrt(jnp.maximum(a[:, 15:16], 1.0))
    ab = jnp.concatenate([a * scale, b * scale], axis=1)
    h = jnp.dot(ab, w1_ref[...], preferred_element_type=jnp.float32)
    h = h + b1_ref[...]
    node = lax.broadcasted_iota(jnp.int32, (_XB, 1), 0) + i * _XB
    valid = node < N
    hmin = jnp.where(valid, h, jnp.inf)
    hmax = jnp.where(valid, h, -jnp.inf)
    cmin_ref[...] = jnp.minimum(cmin_ref[...],
                                jnp.min(hmin, axis=0, keepdims=True))
    cmax_ref[...] = jnp.maximum(cmax_ref[...],
                                jnp.max(hmax, axis=0, keepdims=True))

    @pl.when(i == _XG - 1)
    def _():
        pooled = jnp.maximum(_silu(cmin_ref[...]), _silu(cmax_ref[...]))
        z = _silu(jnp.dot(pooled, w2_ref[...],
                          preferred_element_type=jnp.float32) + b2_ref[...])
        out_ref[...] = jax.nn.sigmoid(
            jnp.dot(z, w3_ref[...], preferred_element_type=jnp.float32)
            + b3_ref[...])


def kernel(features, edge_index, W1, b1, W2, b2, W3, b3):
    epad = jnp.pad(edge_index, ((0, 0), (0, E2 - E)), constant_values=N)
    edges3 = lax.optimization_barrier(epad.reshape(2, ER, SUB))
    epflat = lax.optimization_barrier(epad.reshape(2 * E2))

    degs = _deg_kernel(edges3)
    degs2 = jnp.transpose(degs.reshape(2, N_PAD))

    xbuild = pl.pallas_call(
        _xbuild_body,
        grid=(_XG1,),
        in_specs=[
            pl.BlockSpec((_XB1, F), lambda i: (i, 0)),
            pl.BlockSpec((_XB1, 2), lambda i: (i, 0)),
        ],
        out_specs=pl.BlockSpec((NC, _XB1, 16), lambda i: (0, i, 0)),
        out_shape=jax.ShapeDtypeStruct((NC, N_PAD, 16), jnp.float32),
    )
    tables = xbuild(features, degs2)

    aggP = _agg_kernel(tables, epflat, edges3)

    W1p = jnp.zeros((32, 500), jnp.float32)
    W1p = W1p.at[0:15].set(W1[0:15])
    W1p = W1p.at[16:31].set(W1[15:30])

    head = pl.pallas_call(
        _head_body,
        grid=(_XG,),
        in_specs=[
            pl.BlockSpec((NC, _XB, 16), lambda i: (0, i, 0)),
            pl.BlockSpec((32, 500), lambda i: (0, 0)),
            pl.BlockSpec((1, 500), lambda i: (0, 0)),
            pl.BlockSpec((500, 20), lambda i: (0, 0)),
            pl.BlockSpec((1, 20), lambda i: (0, 0)),
            pl.BlockSpec((20, 4), lambda i: (0, 0)),
            pl.BlockSpec((1, 4), lambda i: (0, 0)),
        ],
        out_specs=pl.BlockSpec((1, 4), lambda i: (0, 0)),
        out_shape=jax.ShapeDtypeStruct((1, 4), jnp.float32),
        scratch_shapes=[
            pltpu.VMEM((1, 500), jnp.float32),
            pltpu.VMEM((1, 500), jnp.float32),
        ],
    )
    return head(aggP, W1p, b1.reshape(1, 500), W2, b2.reshape(1, 20),
                W3, b3.reshape(1, 4))


# speedup vs baseline: 18.1749x; 1.0414x over previous
import functools

import jax
import jax.numpy as jnp
from jax import lax
from jax.experimental import pallas as pl
from jax.experimental.pallas import tpu as pltpu
from jax.experimental.pallas import tpu_sc as plsc

N = 100000
E = 1600000
F = 30
NC = 2
NS = 16
SUB = 128
STRIPE = 6400
N_PAD = NS * STRIPE
PROWS = N_PAD * 16 // 128

E_T2 = 100352
E2 = NS * E_T2
ER = E2 // SUB
R_T2 = E_T2 // SUB
K2 = 2
CH2 = K2 * SUB
NCH2 = R_T2 // K2
NSLOT = 4
NBODY2 = NCH2 // NSLOT

E_T1 = E2 // (NC * NS)
R_T1 = E_T1 // SUB
K1 = 7
CH1 = K1 * SUB
NCH1 = R_T1 // K1
NBODY1 = NCH1 // NSLOT

_mesh = plsc.VectorSubcoreMesh(core_axis_name="c", subcore_axis_name="s")


@functools.partial(
    pl.kernel,
    out_type=jax.ShapeDtypeStruct((NC * N_PAD,), jnp.float32),
    mesh=_mesh,
    compiler_params=pltpu.CompilerParams(use_tc_tiling_on_sc=False),
    scratch_types=[
        pltpu.VMEM((K1, SUB), jnp.int32),
        pltpu.VMEM((K1, SUB), jnp.int32),
        pltpu.VMEM((K1, SUB), jnp.int32),
        pltpu.VMEM((K1, SUB), jnp.int32),
        pltpu.VMEM((SUB,), jnp.float32),
        pltpu.VMEM((STRIPE,), jnp.float32),
        pltpu.VMEM_SHARED((N_PAD,), jnp.float32),
        pltpu.SemaphoreType.DMA,
        pltpu.SemaphoreType.DMA,
        pltpu.SemaphoreType.DMA,
        pltpu.SemaphoreType.DMA,
        pltpu.SemaphoreType.DMA,
    ],
)
def _deg_kernel(edges3, out, i0, i1, i2, i3, ones_v, zbuf, deg_sh,
                l0, l1, l2, l3, ssem):
    c = lax.axis_index("c")
    s = lax.axis_index("s")
    idxs = [i0, i1, i2, i3]
    lsems = [l0, l1, l2, l3]

    def zbody(i, _):
        zbuf[pl.ds(i * 16, 16)] = jnp.zeros((16,), jnp.float32)
        return 0
    lax.fori_loop(0, STRIPE // 16, zbody, 0)
    for i in range(SUB // 16):
        ones_v[pl.ds(i * 16, 16)] = jnp.ones((16,), jnp.float32)
    pltpu.sync_copy(zbuf, deg_sh.at[pl.ds(s * STRIPE, STRIPE)])
    plsc.subcore_barrier()

    row0 = (c * NS + s) * R_T1

    def start_load(chunk, idx, lsem):
        pltpu.async_copy(edges3.at[0, pl.ds(row0 + chunk * K1, K1)], idx, lsem)

    def wait_load(idx, lsem):
        pltpu.make_async_copy(edges3.at[0, pl.ds(0, K1)], idx, lsem).wait()

    def scatter_chunk(idx):
        descs = [pltpu.async_copy(ones_v, deg_sh.at[idx.at[j]], ssem, add=True)
                 for j in range(K1)]
        for d in descs:
            d.wait()

    for t in range(NSLOT):
        start_load(t, idxs[t], lsems[t])

    def body(k, _):
        for t in range(NSLOT):
            wait_load(idxs[t], lsems[t])
            scatter_chunk(idxs[t])

            @pl.when(k < NBODY1 - 1)
            def _():
                start_load(NSLOT * k + t + NSLOT, idxs[t], lsems[t])
        return 0
    lax.fori_loop(0, NBODY1, body, 0)
    plsc.subcore_barrier()
    pltpu.sync_copy(deg_sh.at[pl.ds(s * STRIPE, STRIPE)], zbuf)
    pltpu.sync_copy(zbuf, out.at[pl.ds(c * N_PAD + s * STRIPE, STRIPE)])


@functools.partial(
    pl.kernel,
    out_type=jax.ShapeDtypeStruct((NC, N_PAD, 16), jnp.float32),
    mesh=_mesh,
    compiler_params=pltpu.CompilerParams(use_tc_tiling_on_sc=False),
    scratch_types=[
        pltpu.VMEM((CH2,), jnp.int32),
        pltpu.VMEM((CH2,), jnp.int32),
        pltpu.VMEM((CH2,), jnp.int32),
        pltpu.VMEM((CH2,), jnp.int32),
        pltpu.VMEM((K2, SUB), jnp.int32),
        pltpu.VMEM((K2, SUB), jnp.int32),
        pltpu.VMEM((K2, SUB), jnp.int32),
        pltpu.VMEM((K2, SUB), jnp.int32),
        pltpu.VMEM((CH2, 16), jnp.float32),
        pltpu.VMEM((CH2, 16), jnp.float32),
        pltpu.VMEM((CH2, 16), jnp.float32),
        pltpu.VMEM((CH2, 16), jnp.float32),
        pltpu.VMEM_SHARED((N_PAD, 16), jnp.float32),
        pltpu.SemaphoreType.DMA,
        pltpu.SemaphoreType.DMA,
        pltpu.SemaphoreType.DMA,
        pltpu.SemaphoreType.DMA,
        pltpu.SemaphoreType.DMA,
        pltpu.SemaphoreType.DMA,
        pltpu.SemaphoreType.DMA,
        pltpu.SemaphoreType.DMA,
        pltpu.SemaphoreType.DMA,
    ],
)
def _agg_kernel(tables, epflat, edges3, out, s0, s1, s2, s3, d0, d1, d2, d3,
                r0, r1, r2, r3, agg_sh, g0, g1, g2, g3, l0, l1, l2, l3, ssem):
    c = lax.axis_index("c")
    s = lax.axis_index("s")
    sidxs = [s0, s1, s2, s3]
    didxs = [d0, d1, d2, d3]
    rows = [r0, r1, r2, r3]
    gsems = [g0, g1, g2, g3]
    lsems = [l0, l1, l2, l3]

    def zrow(i, _):
        r0[i] = jnp.zeros((16,), jnp.float32)
        return 0
    lax.fori_loop(0, CH2, zrow, 0)
    nfull = STRIPE // CH2
    for t in range(nfull):
        pltpu.sync_copy(r0, agg_sh.at[pl.ds(s * STRIPE + t * CH2, CH2)])
    plsc.subcore_barrier()

    base = s * E_T2
    row_base = s * R_T2
    tab = tables.at[c]

    def start_load(chunk, sidx, didx, lsem):
        pltpu.async_copy(epflat.at[pl.ds(base + chunk * CH2, CH2)], sidx, lsem)
        pltpu.async_copy(edges3.at[1, pl.ds(row_base + chunk * K2, K2)],
                         didx, lsem)

    def wait_loads(sidx, didx, lsem):
        pltpu.make_async_copy(epflat.at[pl.ds(0, CH2)], sidx, lsem).wait()
        pltpu.make_async_copy(edges3.at[1, pl.ds(0, K2)], didx, lsem).wait()

    def wait_gather(rows_t, gsem):
        pltpu.make_async_copy(tab.at[pl.ds(0, CH2)], rows_t, gsem).wait()

    def scatter_chunk(rows_t, didx):
        descs = [pltpu.async_copy(rows_t.at[pl.ds(j * SUB, SUB)],
                                  agg_sh.at[didx.at[j]], ssem, add=True)
                 for j in range(K2)]
        for d in descs:
            d.wait()

    for t in range(NSLOT):
        start_load(t, sidxs[t], didxs[t], lsems[t])
    for t in range(NSLOT):
        wait_loads(sidxs[t], didxs[t], lsems[t])
        pltpu.async_copy(tab.at[sidxs[t]], rows[t], gsems[t])

    def body(k, _):
        for t in range(NSLOT):
            wait_gather(rows[t], gsems[t])
            scatter_chunk(rows[t], didxs[t])

            @pl.when(k < NBODY2 - 1)
            def _():
                nxt = NSLOT * k + t + NSLOT
                start_load(nxt, sidxs[t], didxs[t], lsems[t])
                wait_loads(sidxs[t], didxs[t], lsems[t])
                pltpu.async_copy(tab.at[sidxs[t]], rows[t], gsems[t])
        return 0
    lax.fori_loop(0, NBODY2, body, 0)
    plsc.subcore_barrier()
    for t in range(nfull):
        pltpu.sync_copy(agg_sh.at[pl.ds(s * STRIPE + t * CH2, CH2)], r0)
        pltpu.sync_copy(r0, out.at[c, pl.ds(s * STRIPE + t * CH2, CH2)])


_XB = 2048
_XG = N_PAD // _XB
_XB1 = 2000
_XG1 = N // _XB1


def _xbuild_body(f_ref, deg_ref, out_ref):
    dT = deg_ref[...]
    d = dT[:, 0:1] + dT[:, 1:2]
    scale = lax.rsqrt(jnp.maximum(d, 1.0))
    x = f_ref[...] * scale
    ones = jnp.ones((_XB1, 1), jnp.float32)
    zeros = jnp.zeros((_XB1, 1), jnp.float32)
    out_ref[0] = jnp.concatenate([x[:, :15], ones], axis=1)
    out_ref[1] = jnp.concatenate([x[:, 15:], zeros], axis=1)


def _silu(x):
    return x * jax.nn.sigmoid(x)


def _head_body(agg_ref, w1_ref, b1_ref, w2_ref, b2_ref, w3_ref, b3_ref,
               out_ref, cmin_ref, cmax_ref):
    i = pl.program_id(0)

    @pl.when(i == 0)
    def _():
        cmin_ref[...] = jnp.full((1, 500), jnp.inf, jnp.float32)
        cmax_ref[...] = jnp.full((1, 500), -jnp.inf, jnp.float32)

    a = agg_ref[0]
    b = agg_ref[1]
    scale = lax.rsqrt(jnp.maximum(a[:, 15:16], 1.0))
    ab = jnp.concatenate([a * scale, b * scale], axis=1)
    h = jnp.dot(ab, w1_ref[...], preferred_element_type=jnp.float32)
    h = h + b1_ref[...]
    node = lax.broadcasted_iota(jnp.int32, (_XB, 1), 0) + i * _XB
    valid = node < N
    hmin = jnp.where(valid, h, jnp.inf)
    hmax = jnp.where(valid, h, -jnp.inf)
    cmin_ref[...] = jnp.minimum(cmin_ref[...],
                                jnp.min(hmin, axis=0, keepdims=True))
    cmax_ref[...] = jnp.maximum(cmax_ref[...],
                                jnp.max(hmax, axis=0, keepdims=True))

    @pl.when(i == _XG - 1)
    def _():
        pooled = jnp.maximum(_silu(cmin_ref[...]), _silu(cmax_ref[...]))
        z = _silu(jnp.dot(pooled, w2_ref[...],
                          preferred_element_type=jnp.float32) + b2_ref[...])
        out_ref[...] = jax.nn.sigmoid(
            jnp.dot(z, w3_ref[...], preferred_element_type=jnp.float32)
            + b3_ref[...])


def kernel(features, edge_index, W1, b1, W2, b2, W3, b3):
    epad = jnp.pad(edge_index, ((0, 0), (0, E2 - E)), constant_values=N)
    edges3 = lax.optimization_barrier(epad.reshape(2, ER, SUB))
    epflat = lax.optimization_barrier(epad.reshape(2 * E2))

    degs = _deg_kernel(edges3)
    degs2 = jnp.transpose(degs.reshape(2, N_PAD))

    xbuild = pl.pallas_call(
        _xbuild_body,
        grid=(_XG1,),
        in_specs=[
            pl.BlockSpec((_XB1, F), lambda i: (i, 0)),
            pl.BlockSpec((_XB1, 2), lambda i: (i, 0)),
        ],
        out_specs=pl.BlockSpec((NC, _XB1, 16), lambda i: (0, i, 0)),
        out_shape=jax.ShapeDtypeStruct((NC, N_PAD, 16), jnp.float32),
    )
    tables = xbuild(features, degs2)

    aggP = _agg_kernel(tables, epflat, edges3)

    W1p = jnp.zeros((32, 500), jnp.float32)
    W1p = W1p.at[0:15].set(W1[0:15])
    W1p = W1p.at[16:31].set(W1[15:30])

    head = pl.pallas_call(
        _head_body,
        grid=(_XG,),
        in_specs=[
            pl.BlockSpec((NC, _XB, 16), lambda i: (0, i, 0)),
            pl.BlockSpec((32, 500), lambda i: (0, 0)),
            pl.BlockSpec((1, 500), lambda i: (0, 0)),
            pl.BlockSpec((500, 20), lambda i: (0, 0)),
            pl.BlockSpec((1, 20), lambda i: (0, 0)),
            pl.BlockSpec((20, 4), lambda i: (0, 0)),
            pl.BlockSpec((1, 4), lambda i: (0, 0)),
        ],
        out_specs=pl.BlockSpec((1, 4), lambda i: (0, 0)),
        out_shape=jax.ShapeDtypeStruct((1, 4), jnp.float32),
        scratch_shapes=[
            pltpu.VMEM((1, 500), jnp.float32),
            pltpu.VMEM((1, 500), jnp.float32),
        ],
    )
    return head(aggP, W1p, b1.reshape(1, 500), W2, b2.reshape(1, 20),
                W3, b3.reshape(1, 4))
